# Initial kernel scaffold; baseline (speedup 1.0000x reference)
import jax
import jax.numpy as jnp
from jax.experimental import pallas as pl


def kernel(atom_fea, nbr_fea, nbr_idx, W_full, b_full, bn1_gamma, bn1_beta, bn2_gamma, bn2_beta):
    raise NotImplementedError("write your pallas kernel here")


# speedup vs baseline: 2.5221x; 2.5221x over previous
import functools

import jax
import jax.numpy as jnp
from jax import lax
from jax.experimental import pallas as pl
from jax.experimental.pallas import tpu as pltpu
from jax.experimental.pallas import tpu_sc as plsc

_EPS = 1e-5


def _gather_rows_sc(table, idx):
    e = idx.shape[0]
    d = table.shape[1]
    info = plsc.get_sparse_core_info()
    nw = info.num_cores * info.num_subcores
    per_w = e // nw
    chunk = 400
    n_ch = per_w // chunk
    mesh = plsc.VectorSubcoreMesh(core_axis_name="c", subcore_axis_name="s")

    @functools.partial(
        pl.kernel,
        mesh=mesh,
        out_type=jax.ShapeDtypeStruct((e, d), jnp.float32),
        scratch_types=[
            pltpu.VMEM((chunk,), jnp.int32),
            pltpu.VMEM((chunk, d), jnp.float32),
            pltpu.SemaphoreType.DMA,
        ],
    )
    def gather_kernel(table_hbm, idx_hbm, out_hbm, idx_v, rows_v, sem):
        wid = lax.axis_index("s") * info.num_cores + lax.axis_index("c")
        base = wid * per_w

        def body(t, carry):
            off = base + t * chunk
            pltpu.sync_copy(idx_hbm.at[pl.ds(off, chunk)], idx_v)
            pltpu.async_copy(table_hbm.at[idx_v], rows_v, sem).wait()
            pltpu.sync_copy(rows_v, out_hbm.at[pl.ds(off, chunk)])
            return carry

        lax.fori_loop(0, n_ch, body, 0)

    return gather_kernel(table, idx)


def _conv_tc(atom_fea, af_g, nbr_flat, w_self, w_nbr, w_edge, b, g1, b1):
    n, dd = atom_fea.shape
    e = af_g.shape[0]
    m = e // n
    dout = w_self.shape[1]
    dn = nbr_flat.shape[1]
    bn = 400
    be = bn * m
    nb = n // bn

    def body(atom_ref, afg_ref, nf_ref, ws_ref, wn_ref, we_ref, b_ref,
             g1_ref, b1_ref, msg_ref, sum_ref, ssq_ref, sc_ref, sh_ref):
        p = pl.program_id(0)
        i = pl.program_id(1)

        @pl.when(jnp.logical_and(p == 0, i == 0))
        def _init():
            sum_ref[...] = jnp.zeros_like(sum_ref)
            ssq_ref[...] = jnp.zeros_like(ssq_ref)

        a_proj = jnp.dot(atom_ref[...], ws_ref[...],
                         preferred_element_type=jnp.float32) + b_ref[...]
        g_proj = jnp.dot(afg_ref[...], wn_ref[...],
                         preferred_element_type=jnp.float32)
        e_proj = jnp.dot(nf_ref[...], we_ref[...],
                         preferred_element_type=jnp.float32)
        z = (g_proj + e_proj).reshape(bn, m, dout) + a_proj[:, None, :]

        @pl.when(p == 0)
        def _stats():
            zf = z.reshape(be, dout)
            sum_ref[...] += jnp.sum(zf, axis=0, keepdims=True)
            ssq_ref[...] += jnp.sum(zf * zf, axis=0, keepdims=True)

        @pl.when(jnp.logical_and(p == 1, i == 0))
        def _finalize():
            cnt = jnp.float32(e)
            mu = sum_ref[...] / cnt
            var = ssq_ref[...] / cnt - mu * mu
            inv = lax.rsqrt(var + _EPS)
            sc_ref[...] = g1_ref[...] * inv
            sh_ref[...] = b1_ref[...] - mu * g1_ref[...] * inv

        @pl.when(p == 1)
        def _apply():
            zn = z * sc_ref[...][None] + sh_ref[...][None]
            f = jax.nn.sigmoid(zn[:, :, :dd])
            c = jax.nn.softplus(zn[:, :, dd:])
            msg_ref[...] = jnp.sum(f * c, axis=1)

    return pl.pallas_call(
        body,
        grid=(2, nb),
        in_specs=[
            pl.BlockSpec((bn, dd), lambda p, i: (i, 0)),
            pl.BlockSpec((be, dd), lambda p, i: (i, 0)),
            pl.BlockSpec((be, dn), lambda p, i: (i, 0)),
            pl.BlockSpec((dd, dout), lambda p, i: (0, 0)),
            pl.BlockSpec((dd, dout), lambda p, i: (0, 0)),
            pl.BlockSpec((dn, dout), lambda p, i: (0, 0)),
            pl.BlockSpec((1, dout), lambda p, i: (0, 0)),
            pl.BlockSpec((1, dout), lambda p, i: (0, 0)),
            pl.BlockSpec((1, dout), lambda p, i: (0, 0)),
        ],
        out_specs=pl.BlockSpec((bn, dd), lambda p, i: (i, 0)),
        out_shape=jax.ShapeDtypeStruct((n, dd), jnp.float32),
        scratch_shapes=[
            pltpu.VMEM((1, dout), jnp.float32),
            pltpu.VMEM((1, dout), jnp.float32),
            pltpu.VMEM((1, dout), jnp.float32),
            pltpu.VMEM((1, dout), jnp.float32),
        ],
    )(atom_fea, af_g, nbr_flat, w_self, w_nbr, w_edge, b, g1, b1)


def _bn2_residual(atom_fea, msg, g2, b2):
    n, dd = atom_fea.shape

    def body(atom_ref, msg_ref, g2_ref, b2_ref, out_ref):
        msgv = msg_ref[...]
        cnt = jnp.float32(n)
        mu = jnp.sum(msgv, axis=0, keepdims=True) / cnt
        zc = msgv - mu
        var = jnp.sum(zc * zc, axis=0, keepdims=True) / cnt
        inv = lax.rsqrt(var + _EPS)
        zn = zc * (g2_ref[...] * inv) + b2_ref[...]
        out_ref[...] = atom_ref[...] + jax.nn.softplus(zn)

    return pl.pallas_call(
        body,
        out_shape=jax.ShapeDtypeStruct((n, dd), jnp.float32),
    )(atom_fea, msg, g2, b2)


def kernel(atom_fea, nbr_fea, nbr_idx, W_full, b_full,
           bn1_gamma, bn1_beta, bn2_gamma, bn2_beta):
    n, m = nbr_idx.shape
    d = atom_fea.shape[1]
    idx_flat = nbr_idx.reshape(-1).astype(jnp.int32)
    af_g = _gather_rows_sc(atom_fea, idx_flat)
    nbr_flat = nbr_fea.reshape(n * m, -1)
    w_self = W_full[:d]
    w_nbr = W_full[d:2 * d]
    w_edge = W_full[2 * d:]
    msg = _conv_tc(atom_fea, af_g, nbr_flat, w_self, w_nbr, w_edge,
                   b_full.reshape(1, -1), bn1_gamma.reshape(1, -1),
                   bn1_beta.reshape(1, -1))
    return _bn2_residual(atom_fea, msg, bn2_gamma.reshape(1, -1),
                         bn2_beta.reshape(1, -1))


# speedup vs baseline: 2.5334x; 1.0045x over previous
import functools

import jax
import jax.numpy as jnp
from jax import lax
from jax.experimental import pallas as pl
from jax.experimental.pallas import tpu as pltpu
from jax.experimental.pallas import tpu_sc as plsc

_EPS = 1e-5


def _gather_rows_sc(table, idx):
    e = idx.shape[0]
    d = table.shape[1]
    info = plsc.get_sparse_core_info()
    nw = info.num_cores * info.num_subcores
    per_w = e // nw
    chunk = 400
    n_ch = per_w // chunk
    mesh = plsc.VectorSubcoreMesh(core_axis_name="c", subcore_axis_name="s")

    @functools.partial(
        pl.kernel,
        mesh=mesh,
        out_type=jax.ShapeDtypeStruct((e, d), table.dtype),
        scratch_types=[
            pltpu.VMEM((chunk,), jnp.int32),
            pltpu.VMEM((chunk, d), table.dtype),
            pltpu.SemaphoreType.DMA,
        ],
    )
    def gather_kernel(table_hbm, idx_hbm, out_hbm, idx_v, rows_v, sem):
        wid = lax.axis_index("s") * info.num_cores + lax.axis_index("c")
        base = wid * per_w

        def body(t, carry):
            off = base + t * chunk
            pltpu.sync_copy(idx_hbm.at[pl.ds(off, chunk)], idx_v)
            pltpu.async_copy(table_hbm.at[idx_v], rows_v, sem).wait()
            pltpu.sync_copy(rows_v, out_hbm.at[pl.ds(off, chunk)])
            return carry

        lax.fori_loop(0, n_ch, body, 0)

    return gather_kernel(table, idx)


def _conv_tc(atom_fea, af_g, nbr_fea, w_self, w_nbr, w_edge, b, g1, b1):
    n, dd = atom_fea.shape
    e = af_g.shape[0]
    m = e // n
    dout = w_self.shape[1]
    dn = nbr_fea.shape[2]
    bn = 400
    be = bn * m
    nb = n // bn

    def body(atom_ref, afg_ref, nf_ref, ws_ref, wn_ref, we_ref, b_ref,
             g1_ref, b1_ref, msg_ref, sum_ref, ssq_ref, sc_ref, sh_ref):
        p = pl.program_id(0)
        i = pl.program_id(1)

        @pl.when(jnp.logical_and(p == 0, i == 0))
        def _init():
            sum_ref[...] = jnp.zeros_like(sum_ref)
            ssq_ref[...] = jnp.zeros_like(ssq_ref)

        a_proj = jnp.dot(atom_ref[...].astype(jnp.bfloat16), ws_ref[...],
                         preferred_element_type=jnp.float32) + b_ref[...]
        g_proj = jnp.dot(afg_ref[...].astype(jnp.bfloat16), wn_ref[...],
                         preferred_element_type=jnp.float32)
        nf = nf_ref[...].reshape(be, dn).astype(jnp.bfloat16)
        e_proj = jnp.dot(nf, we_ref[...], preferred_element_type=jnp.float32)
        z = (g_proj + e_proj).reshape(bn, m, dout) + a_proj[:, None, :]

        @pl.when(p == 0)
        def _stats():
            zf = z.reshape(be, dout)
            sum_ref[...] += jnp.sum(zf, axis=0, keepdims=True)
            ssq_ref[...] += jnp.sum(zf * zf, axis=0, keepdims=True)

        @pl.when(jnp.logical_and(p == 1, i == 0))
        def _finalize():
            cnt = jnp.float32(e)
            mu = sum_ref[...] / cnt
            var = ssq_ref[...] / cnt - mu * mu
            inv = lax.rsqrt(var + _EPS)
            sc_ref[...] = g1_ref[...] * inv
            sh_ref[...] = b1_ref[...] - mu * g1_ref[...] * inv

        @pl.when(p == 1)
        def _apply():
            zn = z * sc_ref[...][None] + sh_ref[...][None]
            f = jax.nn.sigmoid(zn[:, :, :dd])
            c = jax.nn.softplus(zn[:, :, dd:])
            msg_ref[...] = jnp.sum(f * c, axis=1)

    return pl.pallas_call(
        body,
        grid=(2, nb),
        in_specs=[
            pl.BlockSpec((bn, dd), lambda p, i: (i, 0)),
            pl.BlockSpec((be, dd), lambda p, i: (i, 0)),
            pl.BlockSpec((bn, m, dn), lambda p, i: (i, 0, 0)),
            pl.BlockSpec((dd, dout), lambda p, i: (0, 0)),
            pl.BlockSpec((dd, dout), lambda p, i: (0, 0)),
            pl.BlockSpec((dn, dout), lambda p, i: (0, 0)),
            pl.BlockSpec((1, dout), lambda p, i: (0, 0)),
            pl.BlockSpec((1, dout), lambda p, i: (0, 0)),
            pl.BlockSpec((1, dout), lambda p, i: (0, 0)),
        ],
        out_specs=pl.BlockSpec((bn, dd), lambda p, i: (i, 0)),
        out_shape=jax.ShapeDtypeStruct((n, dd), jnp.float32),
        scratch_shapes=[
            pltpu.VMEM((1, dout), jnp.float32),
            pltpu.VMEM((1, dout), jnp.float32),
            pltpu.VMEM((1, dout), jnp.float32),
            pltpu.VMEM((1, dout), jnp.float32),
        ],
    )(atom_fea, af_g, nbr_fea, w_self, w_nbr, w_edge, b, g1, b1)


def _bn2_residual(atom_fea, msg, g2, b2):
    n, dd = atom_fea.shape

    def body(atom_ref, msg_ref, g2_ref, b2_ref, out_ref):
        msgv = msg_ref[...]
        cnt = jnp.float32(n)
        mu = jnp.sum(msgv, axis=0, keepdims=True) / cnt
        zc = msgv - mu
        var = jnp.sum(zc * zc, axis=0, keepdims=True) / cnt
        inv = lax.rsqrt(var + _EPS)
        zn = zc * (g2_ref[...] * inv) + b2_ref[...]
        out_ref[...] = atom_ref[...] + jax.nn.softplus(zn)

    return pl.pallas_call(
        body,
        out_shape=jax.ShapeDtypeStruct((n, dd), jnp.float32),
    )(atom_fea, msg, g2, b2)


def kernel(atom_fea, nbr_fea, nbr_idx, W_full, b_full,
           bn1_gamma, bn1_beta, bn2_gamma, bn2_beta):
    n, m = nbr_idx.shape
    d = atom_fea.shape[1]
    idx_flat = nbr_idx.reshape(-1).astype(jnp.int32)
    af_g = _gather_rows_sc(atom_fea, idx_flat)
    w16 = W_full.astype(jnp.bfloat16)
    w_self = w16[:d]
    w_nbr = w16[d:2 * d]
    w_edge = w16[2 * d:]
    msg = _conv_tc(atom_fea, af_g, nbr_fea, w_self, w_nbr, w_edge,
                   b_full.reshape(1, -1), bn1_gamma.reshape(1, -1),
                   bn1_beta.reshape(1, -1))
    return _bn2_residual(atom_fea, msg, bn2_gamma.reshape(1, -1),
                         bn2_beta.reshape(1, -1))


# speedup vs baseline: 2.6248x; 1.0361x over previous
import functools

import jax
import jax.numpy as jnp
from jax import lax
from jax.experimental import pallas as pl
from jax.experimental.pallas import tpu as pltpu
from jax.experimental.pallas import tpu_sc as plsc

_EPS = 1e-5


def _gather_rows_sc(table, idx):
    e = idx.shape[0]
    d = table.shape[1]
    info = plsc.get_sparse_core_info()
    nw = info.num_cores * info.num_subcores
    per_w = e // nw
    chunk = 400
    n_ch = per_w // chunk
    mesh = plsc.VectorSubcoreMesh(core_axis_name="c", subcore_axis_name="s")

    @functools.partial(
        pl.kernel,
        mesh=mesh,
        out_type=jax.ShapeDtypeStruct((e, d), table.dtype),
        scratch_types=[
            pltpu.VMEM((chunk,), jnp.int32),
            pltpu.VMEM((chunk,), jnp.int32),
            pltpu.VMEM((chunk, d), table.dtype),
            pltpu.VMEM((chunk, d), table.dtype),
            pltpu.SemaphoreType.DMA,
            pltpu.SemaphoreType.DMA,
        ],
    )
    def gather_kernel(table_hbm, idx_hbm, out_hbm, idx_v0, idx_v1,
                      rows_v0, rows_v1, sem_g, sem_s):
        wid = lax.axis_index("s") * info.num_cores + lax.axis_index("c")
        base = wid * per_w
        idx_v = [idx_v0, idx_v1]
        rows_v = [rows_v0, rows_v1]

        pltpu.sync_copy(idx_hbm.at[pl.ds(base, chunk)], idx_v[0])
        pltpu.async_copy(table_hbm.at[idx_v[0]], rows_v[0], sem_g)
        scatter_live = [False, False]
        for t in range(n_ch):
            bsl = t % 2
            if t + 1 < n_ch:
                nsl = (t + 1) % 2
                pltpu.sync_copy(idx_hbm.at[pl.ds(base + (t + 1) * chunk, chunk)],
                                idx_v[nsl])
                if scatter_live[nsl]:
                    pltpu.make_async_copy(rows_v[nsl],
                                          out_hbm.at[pl.ds(base + (t - 1) * chunk, chunk)],
                                          sem_s).wait()
                    scatter_live[nsl] = False
                pltpu.async_copy(table_hbm.at[idx_v[nsl]], rows_v[nsl], sem_g)
            pltpu.make_async_copy(table_hbm.at[idx_v[bsl]], rows_v[bsl],
                                  sem_g).wait()
            pltpu.async_copy(rows_v[bsl],
                             out_hbm.at[pl.ds(base + t * chunk, chunk)], sem_s)
            scatter_live[bsl] = True
        for sl in range(2):
            if scatter_live[sl]:
                pltpu.make_async_copy(rows_v[sl],
                                      out_hbm.at[pl.ds(base, chunk)], sem_s).wait()

    return gather_kernel(table, idx)


def _conv_tc(atom_fea, af_g, nbr_fea, w_cat, g1, b1):
    n, dd = atom_fea.shape
    e = af_g.shape[0]
    m = e // n
    dout = w_cat.shape[1]
    dn = nbr_fea.shape[2]
    dw = af_g.shape[1]
    bn = 400
    be = bn * m
    nb = n // bn
    af_4d = af_g.reshape(nb, 2, be // 2, dw)

    def body(atom_ref, afg_lo_ref, afg_hi_ref, nf_ref, w_ref,
             g1_ref, b1_ref, msg_ref, sum_ref, ssq_ref, wsc_ref, sh_ref):
        p = pl.program_id(0)
        i = pl.program_id(1)

        @pl.when(jnp.logical_and(p == 0, i == 0))
        def _init():
            sum_ref[...] = jnp.zeros_like(sum_ref)
            ssq_ref[...] = jnp.zeros_like(ssq_ref)

        atom_bf = atom_ref[...].astype(jnp.bfloat16)
        atom_rep = jnp.broadcast_to(atom_bf[:, None, :], (bn, m, dd))
        af_bf = jnp.concatenate(
            [afg_lo_ref[...].reshape(be // 2, dd),
             afg_hi_ref[...].reshape(be // 2, dd)],
            axis=0).astype(jnp.bfloat16)
        lhs = jnp.concatenate(
            [atom_rep.reshape(be, dd), af_bf,
             nf_ref[...].reshape(be, dn)], axis=1)

        @pl.when(p == 0)
        def _stats():
            z = jnp.dot(lhs, w_ref[...], preferred_element_type=jnp.float32)
            zf = z.astype(jnp.bfloat16)
            ones_row = jnp.ones((8, be), dtype=jnp.bfloat16)
            sum_ref[...] += jnp.dot(ones_row, zf,
                                    preferred_element_type=jnp.float32)[:1]
            ssq_ref[...] += jnp.dot(ones_row, zf * zf,
                                    preferred_element_type=jnp.float32)[:1]

        @pl.when(jnp.logical_and(p == 1, i == 0))
        def _finalize():
            cnt = jnp.float32(e)
            mu = sum_ref[...] / cnt
            var = ssq_ref[...] / cnt - mu * mu
            inv = lax.rsqrt(var + _EPS)
            sc = g1_ref[...] * inv
            sh_ref[...] = b1_ref[...] - mu * sc
            wsc_ref[...] = (w_ref[...].astype(jnp.float32)
                            * sc).astype(jnp.bfloat16)

        @pl.when(p == 1)
        def _apply():
            zn = jnp.dot(lhs, wsc_ref[...],
                         preferred_element_type=jnp.float32) + sh_ref[...]
            f = jax.nn.sigmoid(zn[:, :dd])
            c = jax.nn.softplus(zn[:, dd:])
            msg_ref[...] = jnp.sum((f * c).reshape(bn, m, dd), axis=1)

    return pl.pallas_call(
        body,
        grid=(2, nb),
        in_specs=[
            pl.BlockSpec((bn, dd), lambda p, i: (i, 0)),
            pl.BlockSpec((1, 1, be // 2, dw), lambda p, i: (i, 0, 0, 0)),
            pl.BlockSpec((1, 1, be // 2, dw), lambda p, i: (i, 1, 0, 0)),
            pl.BlockSpec((bn, m, dn), lambda p, i: (i, 0, 0)),
            pl.BlockSpec((2 * dd + dn, dout), lambda p, i: (0, 0)),
            pl.BlockSpec((1, dout), lambda p, i: (0, 0)),
            pl.BlockSpec((1, dout), lambda p, i: (0, 0)),
        ],
        out_specs=pl.BlockSpec((bn, dd), lambda p, i: (i, 0)),
        out_shape=jax.ShapeDtypeStruct((n, dd), jnp.float32),
        scratch_shapes=[
            pltpu.VMEM((1, dout), jnp.float32),
            pltpu.VMEM((1, dout), jnp.float32),
            pltpu.VMEM((2 * dd + dn, dout), jnp.bfloat16),
            pltpu.VMEM((1, dout), jnp.float32),
        ],
    )(atom_fea, af_4d, af_4d, nbr_fea, w_cat, g1, b1)


def _bn2_residual(atom_fea, msg, g2, b2):
    n, dd = atom_fea.shape

    def body(atom_ref, msg_ref, g2_ref, b2_ref, out_ref):
        msgv = msg_ref[...]
        cnt = jnp.float32(n)
        mu = jnp.sum(msgv, axis=0, keepdims=True) / cnt
        zc = msgv - mu
        var = jnp.sum(zc * zc, axis=0, keepdims=True) / cnt
        inv = lax.rsqrt(var + _EPS)
        zn = zc * (g2_ref[...] * inv) + b2_ref[...]
        out_ref[...] = atom_ref[...] + jax.nn.softplus(zn)

    return pl.pallas_call(
        body,
        out_shape=jax.ShapeDtypeStruct((n, dd), jnp.float32),
    )(atom_fea, msg, g2, b2)


def kernel(atom_fea, nbr_fea, nbr_idx, W_full, b_full,
           bn1_gamma, bn1_beta, bn2_gamma, bn2_beta):
    n, m = nbr_idx.shape
    d = atom_fea.shape[1]
    idx_flat = nbr_idx.reshape(-1).astype(jnp.int32)
    af_g = _gather_rows_sc(atom_fea, idx_flat)
    del b_full
    msg = _conv_tc(atom_fea, af_g, nbr_fea.astype(jnp.bfloat16),
                   W_full.astype(jnp.bfloat16),
                   bn1_gamma.reshape(1, -1), bn1_beta.reshape(1, -1))
    return _bn2_residual(atom_fea, msg, bn2_gamma.reshape(1, -1),
                         bn2_beta.reshape(1, -1))


# speedup vs baseline: 2.7595x; 1.0513x over previous
import functools

import jax
import jax.numpy as jnp
from jax import lax
from jax.experimental import pallas as pl
from jax.experimental.pallas import tpu as pltpu
from jax.experimental.pallas import tpu_sc as plsc

_EPS = 1e-5
_BN = 200


def _gather_rows_sc(table, idx):
    e = idx.shape[0]
    d = table.shape[1]
    info = plsc.get_sparse_core_info()
    nw = info.num_cores * info.num_subcores
    per_w = e // nw
    chunk = 400
    n_ch = per_w // chunk
    mesh = plsc.VectorSubcoreMesh(core_axis_name="c", subcore_axis_name="s")

    @functools.partial(
        pl.kernel,
        mesh=mesh,
        out_type=jax.ShapeDtypeStruct((e, d), table.dtype),
        scratch_types=[
            pltpu.VMEM((chunk,), jnp.int32),
            pltpu.VMEM((chunk,), jnp.int32),
            pltpu.VMEM((chunk, d), table.dtype),
            pltpu.VMEM((chunk, d), table.dtype),
            pltpu.SemaphoreType.DMA,
            pltpu.SemaphoreType.DMA,
        ],
    )
    def gather_kernel(table_hbm, idx_hbm, out_hbm, idx_v0, idx_v1,
                      rows_v0, rows_v1, sem_g, sem_s):
        wid = lax.axis_index("s") * info.num_cores + lax.axis_index("c")
        base = wid * per_w
        idx_v = [idx_v0, idx_v1]
        rows_v = [rows_v0, rows_v1]

        pltpu.sync_copy(idx_hbm.at[pl.ds(base, chunk)], idx_v[0])
        pltpu.async_copy(table_hbm.at[idx_v[0]], rows_v[0], sem_g)
        scatter_live = [False, False]
        for t in range(n_ch):
            bsl = t % 2
            if t + 1 < n_ch:
                nsl = (t + 1) % 2
                pltpu.sync_copy(idx_hbm.at[pl.ds(base + (t + 1) * chunk, chunk)],
                                idx_v[nsl])
                if scatter_live[nsl]:
                    pltpu.make_async_copy(rows_v[nsl],
                                          out_hbm.at[pl.ds(base, chunk)],
                                          sem_s).wait()
                    scatter_live[nsl] = False
                pltpu.async_copy(table_hbm.at[idx_v[nsl]], rows_v[nsl], sem_g)
            pltpu.make_async_copy(table_hbm.at[idx_v[bsl]], rows_v[bsl],
                                  sem_g).wait()
            pltpu.async_copy(rows_v[bsl],
                             out_hbm.at[pl.ds(base + t * chunk, chunk)], sem_s)
            scatter_live[bsl] = True
        for sl in range(2):
            if scatter_live[sl]:
                pltpu.make_async_copy(rows_v[sl],
                                      out_hbm.at[pl.ds(base, chunk)], sem_s).wait()

    return gather_kernel(table, idx)


def _build_lhs(atom_ref, af_ref, nf_ref, bn, m, dd, dn):
    be = bn * m
    atom_bf = atom_ref[...].astype(jnp.bfloat16)
    atom_rep = jnp.broadcast_to(atom_bf[:, None, :], (bn, m, dd))
    return jnp.concatenate(
        [atom_rep.reshape(be, dd),
         af_ref[...].astype(jnp.bfloat16),
         nf_ref[...].reshape(be, dn)], axis=1)


def _stats_tc(atom_fea, af_h, nbr_fea, w_cat, node_off):
    n, dd = atom_fea.shape
    eh = af_h.shape[0]
    m = nbr_fea.shape[1]
    dn = nbr_fea.shape[2]
    dout = w_cat.shape[1]
    bn = _BN
    be = bn * m
    nbh = eh // be
    ob = node_off // bn

    def body(atom_ref, afg_ref, nf_ref, w_ref, out_ref, sum_ref, ssq_ref):
        i = pl.program_id(0)

        @pl.when(i == 0)
        def _init():
            sum_ref[...] = jnp.zeros_like(sum_ref)
            ssq_ref[...] = jnp.zeros_like(ssq_ref)

        lhs = _build_lhs(atom_ref, afg_ref, nf_ref, bn, m, dd, dn)
        z = jnp.dot(lhs, w_ref[...], preferred_element_type=jnp.float32)
        zf = z.astype(jnp.bfloat16)
        ones_row = jnp.ones((8, be), dtype=jnp.bfloat16)
        sum_ref[...] += jnp.dot(ones_row, zf,
                                preferred_element_type=jnp.float32)[:1]
        ssq_ref[...] += jnp.dot(ones_row, zf * zf,
                                preferred_element_type=jnp.float32)[:1]

        @pl.when(i == nbh - 1)
        def _emit():
            out_ref[0:1, :] = sum_ref[...]
            out_ref[1:2, :] = ssq_ref[...]

    return pl.pallas_call(
        body,
        grid=(nbh,),
        in_specs=[
            pl.BlockSpec((bn, dd), lambda i: (ob + i, 0)),
            pl.BlockSpec((be, dd), lambda i: (i, 0)),
            pl.BlockSpec((bn, m, dn), lambda i: (ob + i, 0, 0)),
            pl.BlockSpec((2 * dd + dn, dout), lambda i: (0, 0)),
        ],
        out_specs=pl.BlockSpec((2, dout), lambda i: (0, 0)),
        out_shape=jax.ShapeDtypeStruct((2, dout), jnp.float32),
        scratch_shapes=[
            pltpu.VMEM((1, dout), jnp.float32),
            pltpu.VMEM((1, dout), jnp.float32),
        ],
    )(atom_fea, af_h, nbr_fea, w_cat)


def _apply_tc(atom_fea, af_a, af_b, nbr_fea, w_cat, g1, b1, stats_a, stats_b):
    n, dd = atom_fea.shape
    e = af_a.shape[0] + af_b.shape[0]
    m = nbr_fea.shape[1]
    dn = nbr_fea.shape[2]
    dout = w_cat.shape[1]
    bn = _BN
    be = bn * m
    nb = n // bn
    nba = af_a.shape[0] // be

    def body(atom_ref, afa_ref, afb_ref, nf_ref, w_ref, g1_ref, b1_ref,
             sa_ref, sb_ref, msg_ref, wsc_ref, sh_ref):
        i = pl.program_id(0)

        @pl.when(i == 0)
        def _finalize():
            cnt = jnp.float32(e)
            tot = sa_ref[...] + sb_ref[...]
            mu = tot[0:1, :] / cnt
            var = tot[1:2, :] / cnt - mu * mu
            inv = lax.rsqrt(var + _EPS)
            sc = g1_ref[...] * inv
            sh_ref[...] = b1_ref[...] - mu * sc
            wsc_ref[...] = (w_ref[...].astype(jnp.float32)
                            * sc).astype(jnp.bfloat16)

        def gate(af_ref):
            lhs = _build_lhs(atom_ref, af_ref, nf_ref, bn, m, dd, dn)
            zn = jnp.dot(lhs, wsc_ref[...],
                         preferred_element_type=jnp.float32) + sh_ref[...]
            f = jax.nn.sigmoid(zn[:, :dd])
            c = jax.nn.softplus(zn[:, dd:])
            msg_ref[...] = jnp.sum((f * c).reshape(bn, m, dd), axis=1)

        @pl.when(i < nba)
        def _half_a():
            gate(afa_ref)

        @pl.when(i >= nba)
        def _half_b():
            gate(afb_ref)

    return pl.pallas_call(
        body,
        grid=(nb,),
        in_specs=[
            pl.BlockSpec((bn, dd), lambda i: (i, 0)),
            pl.BlockSpec((be, dd), lambda i: (jnp.minimum(i, nba - 1), 0)),
            pl.BlockSpec((be, dd), lambda i: (jnp.maximum(i - nba, 0), 0)),
            pl.BlockSpec((bn, m, dn), lambda i: (i, 0, 0)),
            pl.BlockSpec((2 * dd + dn, dout), lambda i: (0, 0)),
            pl.BlockSpec((1, dout), lambda i: (0, 0)),
            pl.BlockSpec((1, dout), lambda i: (0, 0)),
            pl.BlockSpec((2, dout), lambda i: (0, 0)),
            pl.BlockSpec((2, dout), lambda i: (0, 0)),
        ],
        out_specs=pl.BlockSpec((bn, dd), lambda i: (i, 0)),
        out_shape=jax.ShapeDtypeStruct((n, dd), jnp.float32),
        scratch_shapes=[
            pltpu.VMEM((2 * dd + dn, dout), jnp.bfloat16),
            pltpu.VMEM((1, dout), jnp.float32),
        ],
    )(atom_fea, af_a, af_b, nbr_fea, w_cat, g1, b1, stats_a, stats_b)


def _bn2_residual(atom_fea, msg, g2, b2):
    n, dd = atom_fea.shape

    def body(atom_ref, msg_ref, g2_ref, b2_ref, out_ref):
        msgv = msg_ref[...]
        cnt = jnp.float32(n)
        mu = jnp.sum(msgv, axis=0, keepdims=True) / cnt
        zc = msgv - mu
        var = jnp.sum(zc * zc, axis=0, keepdims=True) / cnt
        inv = lax.rsqrt(var + _EPS)
        zn = zc * (g2_ref[...] * inv) + b2_ref[...]
        out_ref[...] = atom_ref[...] + jax.nn.softplus(zn)

    return pl.pallas_call(
        body,
        out_shape=jax.ShapeDtypeStruct((n, dd), jnp.float32),
    )(atom_fea, msg, g2, b2)


def kernel(atom_fea, nbr_fea, nbr_idx, W_full, b_full,
           bn1_gamma, bn1_beta, bn2_gamma, bn2_beta):
    n, m = nbr_idx.shape
    d = atom_fea.shape[1]
    idx_flat = nbr_idx.reshape(-1).astype(jnp.int32)
    na = (n * 13) // 25
    ea = na * m
    af_a = _gather_rows_sc(atom_fea, idx_flat[:ea])
    af_b = _gather_rows_sc(atom_fea, idx_flat[ea:])
    del b_full
    nbr16 = nbr_fea.astype(jnp.bfloat16)
    w16 = W_full.astype(jnp.bfloat16)
    stats_a = _stats_tc(atom_fea, af_a, nbr16, w16, 0)
    stats_b = _stats_tc(atom_fea, af_b, nbr16, w16, na)
    msg = _apply_tc(atom_fea, af_a, af_b, nbr16, w16,
                    bn1_gamma.reshape(1, -1), bn1_beta.reshape(1, -1),
                    stats_a, stats_b)
    return _bn2_residual(atom_fea, msg, bn2_gamma.reshape(1, -1),
                         bn2_beta.reshape(1, -1))
